# Initial kernel scaffold; baseline (speedup 1.0000x reference)
import jax
import jax.numpy as jnp
from jax.experimental import pallas as pl


def kernel(indices, offsets, table):
    raise NotImplementedError("write your pallas kernel here")


# speedup vs baseline: 51.3536x; 51.3536x over previous
import functools

import jax
import jax.numpy as jnp
from jax import lax
from jax.experimental import pallas as pl
from jax.experimental.pallas import tpu as pltpu
from jax.experimental.pallas import tpu_sc as plsc

DIM = 32
BAGS = 106496
N_IDX = 425984

NUM_CORES = 2
NUM_SUBCORES = 16
NW = NUM_CORES * NUM_SUBCORES
BPW = BAGS // NW
C = 256
GSUB = 128
HALF = 16


def _sload(ref, i):
    return ref[pl.ds(i, HALF)][0]


def _upper_bound(off_v, lo0, val):
    def step(_, lohi):
        lo, hi = lohi
        mid = (lo + hi) // 2
        go = (lo < hi) & (_sload(off_v, mid + 1) <= val)
        return (jnp.where(go, mid + 1, lo), jnp.where(go, hi, mid))
    lo, _ = lax.fori_loop(0, 12, step, (lo0, jnp.int32(BPW)))
    return lo


def _pooled_body(idx_hbm, off_hbm, table_hbm, out_hbm,
                 off_v, idx0_v, idx1_v, rows_v, out_v, sem):
    w = lax.axis_index("s") * NUM_CORES + lax.axis_index("c")
    obase = w * BPW
    pltpu.sync_copy(off_hbm.at[pl.ds(obase, BPW + 24)], off_v)
    start = _sload(off_v, 0)
    end = _sload(off_v, BPW)
    astart = (start // 8) * 8
    nch = jnp.maximum((end - astart + C - 1) // C, 1)

    zero = jnp.zeros((HALF,), jnp.float32)

    nzero = _upper_bound(off_v, jnp.int32(0), astart)

    def zero_body(b, _):
        out_v[b, 0:HALF] = zero
        out_v[b, HALF:DIM] = zero
        return 0
    lax.fori_loop(0, nzero, zero_body, 0)

    def chunk_body(g, carry):
        b_cur, acc_lo, acc_hi, last_lo, last_hi = carry
        p0 = astart + g * C
        pltpu.sync_copy(idx_hbm.at[pl.ds(p0, GSUB)], idx0_v)
        pltpu.sync_copy(idx_hbm.at[pl.ds(p0 + GSUB, GSUB)], idx1_v)
        cp0 = pltpu.async_copy(table_hbm.at[idx0_v],
                               rows_v.at[pl.ds(0, GSUB)], sem)
        cp1 = pltpu.async_copy(table_hbm.at[idx1_v],
                               rows_v.at[pl.ds(GSUB, GSUB)], sem)
        cp0.wait()
        cp1.wait()

        def row_body(r, rc):
            a_lo, a_hi = rc
            p = p0 + r
            ok = (p >= start) & (p < end)
            a_lo = a_lo + jnp.where(ok, rows_v[r, 0:HALF], zero)
            a_hi = a_hi + jnp.where(ok, rows_v[r, HALF:DIM], zero)
            rows_v[r, 0:HALF] = a_lo
            rows_v[r, HALF:DIM] = a_hi
            return (a_lo, a_hi)
        acc_lo, acc_hi = lax.fori_loop(0, C, row_body, (acc_lo, acc_hi))

        b_new = _upper_bound(off_v, b_cur, p0 + C)

        def emit_body(b, ec):
            l_lo, l_hi = ec
            r = _sload(off_v, b + 1) - p0 - 1
            pe_lo = rows_v[r, 0:HALF]
            pe_hi = rows_v[r, HALF:DIM]
            out_v[b, 0:HALF] = pe_lo - l_lo
            out_v[b, HALF:DIM] = pe_hi - l_hi
            return (pe_lo, pe_hi)
        last_lo, last_hi = lax.fori_loop(b_cur, b_new, emit_body,
                                         (last_lo, last_hi))
        return (b_new, acc_lo, acc_hi, last_lo, last_hi)

    lax.fori_loop(0, nch, chunk_body, (nzero, zero, zero, zero, zero))
    pltpu.sync_copy(out_v, out_hbm.at[pl.ds(obase, BPW)])


_pooled = functools.partial(
    pl.kernel,
    out_type=jax.ShapeDtypeStruct((BAGS, DIM), jnp.float32),
    mesh=plsc.VectorSubcoreMesh(core_axis_name="c", subcore_axis_name="s"),
    compiler_params=pltpu.CompilerParams(use_tc_tiling_on_sc=False),
    scratch_types=[
        pltpu.VMEM((BPW + 24,), jnp.int32),
        pltpu.VMEM((GSUB,), jnp.int32),
        pltpu.VMEM((GSUB,), jnp.int32),
        pltpu.VMEM((C, DIM), jnp.float32),
        pltpu.VMEM((BPW, DIM), jnp.float32),
        pltpu.SemaphoreType.DMA,
    ],
)(_pooled_body)


def kernel(indices, offsets, table):
    idx = indices.astype(jnp.int32)
    off = offsets.astype(jnp.int32)
    idx = jnp.concatenate([idx, jnp.zeros((C + 8,), jnp.int32)])
    off = jnp.concatenate([off, jnp.full((24,), N_IDX, jnp.int32)])
    return _pooled(idx, off, table)


# speedup vs baseline: 62.9008x; 1.2249x over previous
import functools

import jax
import jax.numpy as jnp
from jax import lax
from jax.experimental import pallas as pl
from jax.experimental.pallas import tpu as pltpu
from jax.experimental.pallas import tpu_sc as plsc

DIM = 32
BAGS = 106496
N_IDX = 425984

NUM_CORES = 2
NUM_SUBCORES = 16
NW = NUM_CORES * NUM_SUBCORES
BPW = BAGS // NW
C = 256
GSUB = 128
HALF = 16
UNROLL = 8


def _sload(ref, i):
    return ref[pl.ds(i, HALF)][0]


def _upper_bound(off_v, lo0, val):
    def step(_, lohi):
        lo, hi = lohi
        mid = (lo + hi) // 2
        go = (lo < hi) & (_sload(off_v, mid + 1) <= val)
        return (jnp.where(go, mid + 1, lo), jnp.where(go, hi, mid))
    lo, _ = lax.fori_loop(0, 12, step, (lo0, jnp.int32(BPW)))
    return lo


def _pooled_body(idx_hbm, off_hbm, table_hbm, out_hbm,
                 off_v, ia0, ib0, ia1, ib1, rows0, rows1, out_v,
                 semi0, semi1, semg0, semg1):
    w = lax.axis_index("s") * NUM_CORES + lax.axis_index("c")
    obase = w * BPW
    pltpu.sync_copy(off_hbm.at[pl.ds(obase, BPW + 24)], off_v)
    start = _sload(off_v, 0)
    end = _sload(off_v, BPW)
    astart = (start // 8) * 8
    nch = jnp.maximum((end - astart + C - 1) // C, 1)
    nch2 = ((nch + 1) // 2) * 2

    idx_bufs = ((ia0, ib0), (ia1, ib1))
    row_bufs = (rows0, rows1)
    semi = (semi0, semi1)
    semg = (semg0, semg1)
    zero = jnp.zeros((HALF,), jnp.float32)

    def stage_idx(g, s):
        p0 = astart + g * C
        a = pltpu.async_copy(idx_hbm.at[pl.ds(p0, GSUB)], idx_bufs[s][0],
                             semi[s])
        b = pltpu.async_copy(idx_hbm.at[pl.ds(p0 + GSUB, GSUB)],
                             idx_bufs[s][1], semi[s])
        del a, b

    def stage_gather(g, s):
        pltpu.make_async_copy(idx_hbm.at[pl.ds(0, GSUB)], idx_bufs[s][0],
                              semi[s]).wait()
        pltpu.make_async_copy(idx_hbm.at[pl.ds(0, GSUB)], idx_bufs[s][1],
                              semi[s]).wait()
        a = pltpu.async_copy(table_hbm.at[idx_bufs[s][0]],
                             row_bufs[s].at[pl.ds(0, GSUB)], semg[s])
        b = pltpu.async_copy(table_hbm.at[idx_bufs[s][1]],
                             row_bufs[s].at[pl.ds(GSUB, GSUB)], semg[s])
        del a, b

    def process(g, s, carry):
        b_cur, acc_lo, acc_hi, last_lo, last_hi = carry
        rows_v = row_bufs[s]
        p0 = astart + g * C
        pltpu.make_async_copy(
            table_hbm.at[idx_bufs[s][0]],
            rows_v.at[pl.ds(0, GSUB)], semg[s]).wait()
        pltpu.make_async_copy(
            table_hbm.at[idx_bufs[s][1]],
            rows_v.at[pl.ds(GSUB, GSUB)], semg[s]).wait()

        hz = jnp.clip(start - p0, 0, C)
        tz = jnp.clip(end - p0, 0, C)

        def zero_body(r, _):
            rows_v[r, 0:HALF] = zero
            rows_v[r, HALF:DIM] = zero
            return 0
        lax.fori_loop(0, hz, zero_body, 0)
        lax.fori_loop(tz, C, zero_body, 0)

        def row_body(u, rc):
            a_lo, a_hi = rc
            base = u * UNROLL
            for k in range(UNROLL):
                r = base + k
                a_lo = a_lo + rows_v[r, 0:HALF]
                a_hi = a_hi + rows_v[r, HALF:DIM]
                rows_v[r, 0:HALF] = a_lo
                rows_v[r, HALF:DIM] = a_hi
            return (a_lo, a_hi)
        acc_lo, acc_hi = lax.fori_loop(0, C // UNROLL, row_body,
                                       (acc_lo, acc_hi))

        b_new = _upper_bound(off_v, b_cur, p0 + C)

        def emit_body(b, ec):
            l_lo, l_hi = ec
            r = _sload(off_v, b + 1) - p0 - 1
            pe_lo = rows_v[r, 0:HALF]
            pe_hi = rows_v[r, HALF:DIM]
            out_v[b, 0:HALF] = pe_lo - l_lo
            out_v[b, HALF:DIM] = pe_hi - l_hi
            return (pe_lo, pe_hi)
        last_lo, last_hi = lax.fori_loop(b_cur, b_new, emit_body,
                                         (last_lo, last_hi))
        return (b_new, acc_lo, acc_hi, last_lo, last_hi)

    nzero = _upper_bound(off_v, jnp.int32(0), astart)

    def zero_out_body(b, _):
        out_v[b, 0:HALF] = zero
        out_v[b, HALF:DIM] = zero
        return 0
    lax.fori_loop(0, nzero, zero_out_body, 0)

    stage_idx(0, 0)
    stage_gather(0, 0)
    stage_idx(1, 1)

    def pair_body(h, carry):
        g0 = h * 2
        stage_gather(g0 + 1, 1)

        @pl.when(g0 + 2 < nch2)
        def _():
            stage_idx(g0 + 2, 0)
        carry = process(g0, 0, carry)

        @pl.when(g0 + 3 < nch2)
        def _():
            stage_idx(g0 + 3, 1)

        @pl.when(g0 + 2 < nch2)
        def _():
            stage_gather(g0 + 2, 0)
        carry = process(g0 + 1, 1, carry)
        return carry

    lax.fori_loop(0, nch2 // 2, pair_body,
                  (nzero, zero, zero, zero, zero))
    pltpu.sync_copy(out_v, out_hbm.at[pl.ds(obase, BPW)])


_pooled = functools.partial(
    pl.kernel,
    out_type=jax.ShapeDtypeStruct((BAGS, DIM), jnp.float32),
    mesh=plsc.VectorSubcoreMesh(core_axis_name="c", subcore_axis_name="s"),
    compiler_params=pltpu.CompilerParams(use_tc_tiling_on_sc=False),
    scratch_types=[
        pltpu.VMEM((BPW + 24,), jnp.int32),
        pltpu.VMEM((GSUB,), jnp.int32),
        pltpu.VMEM((GSUB,), jnp.int32),
        pltpu.VMEM((GSUB,), jnp.int32),
        pltpu.VMEM((GSUB,), jnp.int32),
        pltpu.VMEM((C, DIM), jnp.float32),
        pltpu.VMEM((C, DIM), jnp.float32),
        pltpu.VMEM((BPW, DIM), jnp.float32),
        pltpu.SemaphoreType.DMA,
        pltpu.SemaphoreType.DMA,
        pltpu.SemaphoreType.DMA,
        pltpu.SemaphoreType.DMA,
    ],
)(_pooled_body)


def kernel(indices, offsets, table):
    idx = indices.astype(jnp.int32)
    off = offsets.astype(jnp.int32)
    idx = jnp.concatenate([idx, jnp.zeros((2 * C + 8,), jnp.int32)])
    off = jnp.concatenate([off, jnp.full((24,), N_IDX, jnp.int32)])
    return _pooled(idx, off, table)


# speedup vs baseline: 67.2789x; 1.0696x over previous
import functools

import jax
import jax.numpy as jnp
from jax import lax
from jax.experimental import pallas as pl
from jax.experimental.pallas import tpu as pltpu
from jax.experimental.pallas import tpu_sc as plsc

DIM = 32
BAGS = 106496
N_IDX = 425984

NUM_CORES = 2
NUM_SUBCORES = 16
NW = NUM_CORES * NUM_SUBCORES
BPW = BAGS // NW
SLAB = BPW + 8
C = 256
GSUB = 128
HALF = 16
NGRP = C // HALF


def _sload(ref, i):
    return ref[pl.ds(i, HALF)][0]


def _upper_bound(off_v, lo0, val):
    def step(_, lohi):
        lo, hi = lohi
        mid = (lo + hi) // 2
        go = (lo < hi) & (_sload(off_v, mid + 1) <= val)
        return (jnp.where(go, mid + 1, lo), jnp.where(go, hi, mid))
    lo, _ = lax.fori_loop(0, 12, step, (lo0, jnp.int32(BPW)))
    return lo


def _pooled_body(idx_hbm, off_hbm, table_hbm, out_hbm,
                 off_v, ia0, ib0, ia1, ib1, rows0, rows1, m_v, sg0, sg1,
                 acc_sh, semi0, semi1, semg0, semg1, sems0, sems1, semz):
    w = lax.axis_index("s") * NUM_CORES + lax.axis_index("c")
    sid = lax.axis_index("s")
    obase = w * BPW
    sbase = sid * SLAB
    dummy = sbase + BPW
    pltpu.sync_copy(off_hbm.at[pl.ds(obase, BPW + 24)], off_v)
    start = _sload(off_v, 0)
    end = _sload(off_v, BPW)
    astart = (start // 8) * 8
    nch = jnp.maximum((end - astart + C - 1) // C, 1)
    nch2 = ((nch + 1) // 2) * 2

    idx_bufs = ((ia0, ib0), (ia1, ib1))
    row_bufs = (rows0, rows1)
    seg_bufs = (sg0, sg1)
    semi = (semi0, semi1)
    semg = (semg0, semg1)
    sems = (sems0, sems1)
    zero = jnp.zeros((HALF,), jnp.float32)
    zero_i = jnp.zeros((HALF,), jnp.int32)
    lane = jnp.arange(HALF, dtype=jnp.int32)

    def zrow(r, _):
        rows0[r, 0:HALF] = zero
        rows0[r, HALF:DIM] = zero
        return 0
    lax.fori_loop(0, C, zrow, 0)
    for s in range(2):
        for j in range(2):
            for k in range(GSUB // HALF):
                seg_bufs[s][j, pl.ds(k * HALF, HALF)] = (
                    jnp.full((HALF,), 0, jnp.int32) + dummy)
    nzc = BPW // C
    for k in range(nzc):
        pltpu.async_copy(rows0, acc_sh.at[pl.ds(sbase + k * C, C)], semz)
    pltpu.async_copy(rows0.at[pl.ds(0, 8)],
                     acc_sh.at[pl.ds(sbase + BPW, 8)], semz)
    for k in range(nzc):
        pltpu.make_async_copy(rows0,
                              acc_sh.at[pl.ds(sbase + k * C, C)], semz).wait()
    pltpu.make_async_copy(rows0.at[pl.ds(0, 8)],
                          acc_sh.at[pl.ds(sbase + BPW, 8)], semz).wait()

    for s in range(2):
        pltpu.async_copy(rows0.at[pl.ds(0, GSUB)],
                         acc_sh.at[seg_bufs[s].at[0]], sems[s], add=True)
        pltpu.async_copy(rows0.at[pl.ds(GSUB, GSUB)],
                         acc_sh.at[seg_bufs[s].at[1]], sems[s], add=True)

    def stage_idx(g, s):
        p0 = astart + g * C
        pltpu.async_copy(idx_hbm.at[pl.ds(p0, GSUB)], idx_bufs[s][0], semi[s])
        pltpu.async_copy(idx_hbm.at[pl.ds(p0 + GSUB, GSUB)],
                         idx_bufs[s][1], semi[s])

    def stage_gather(g, s):
        pltpu.make_async_copy(row_bufs[s].at[pl.ds(0, GSUB)],
                              acc_sh.at[seg_bufs[s].at[0]], sems[s]).wait()
        pltpu.make_async_copy(row_bufs[s].at[pl.ds(GSUB, GSUB)],
                              acc_sh.at[seg_bufs[s].at[1]], sems[s]).wait()
        pltpu.make_async_copy(idx_hbm.at[pl.ds(0, GSUB)], idx_bufs[s][0],
                              semi[s]).wait()
        pltpu.make_async_copy(idx_hbm.at[pl.ds(0, GSUB)], idx_bufs[s][1],
                              semi[s]).wait()
        pltpu.async_copy(table_hbm.at[idx_bufs[s][0]],
                         row_bufs[s].at[pl.ds(0, GSUB)], semg[s])
        pltpu.async_copy(table_hbm.at[idx_bufs[s][1]],
                         row_bufs[s].at[pl.ds(GSUB, GSUB)], semg[s])

    def process(g, s, b_cur):
        rows_v = row_bufs[s]
        seg_v = seg_bufs[s]
        p0 = astart + g * C
        pltpu.make_async_copy(
            table_hbm.at[idx_bufs[s][0]],
            rows_v.at[pl.ds(0, GSUB)], semg[s]).wait()
        pltpu.make_async_copy(
            table_hbm.at[idx_bufs[s][1]],
            rows_v.at[pl.ds(GSUB, GSUB)], semg[s]).wait()

        b_new = _upper_bound(off_v, b_cur, p0 + C - 1)
        for j in range(NGRP):
            m_v[pl.ds(j * HALF, HALF)] = zero_i
        ng = (b_new - b_cur + HALF - 1) // HALF

        def mark(gi, _):
            b = b_cur + gi * HALF
            e = off_v[pl.ds(b + 1, HALF)]
            enext = off_v[pl.ds(b + 2, HALF)]
            valid = ((lane + b) < b_new) & (e < enext)
            q = jnp.clip(e - p0, 0, C - 1)
            plsc.store_scatter(m_v, [q], lane + (b + 1), mask=valid)
            return 0
        lax.fori_loop(0, ng, mark, 0)

        carry = b_cur
        for j in range(NGRP):
            mv = m_v[pl.ds(j * HALF, HALF)]
            cm = plsc.cummax(mv)
            seg = jnp.maximum(cm, carry)
            carry = seg[HALF - 1]
            pos = (p0 + j * HALF) + lane
            ok = (pos >= start) & (pos < end)
            segc = jnp.where(ok, seg + sbase, dummy)
            seg_v[j // (GSUB // HALF),
                  pl.ds((j % (GSUB // HALF)) * HALF, HALF)] = segc

        pltpu.async_copy(rows_v.at[pl.ds(0, GSUB)],
                         acc_sh.at[seg_v.at[0]], sems[s], add=True)
        pltpu.async_copy(rows_v.at[pl.ds(GSUB, GSUB)],
                         acc_sh.at[seg_v.at[1]], sems[s], add=True)
        return b_new

    stage_idx(0, 0)
    stage_gather(0, 0)
    stage_idx(1, 1)

    def pair_body(h, b_cur):
        g0 = h * 2
        stage_gather(g0 + 1, 1)

        @pl.when(g0 + 2 < nch2)
        def _():
            stage_idx(g0 + 2, 0)
        b_cur = process(g0, 0, b_cur)

        @pl.when(g0 + 3 < nch2)
        def _():
            stage_idx(g0 + 3, 1)

        @pl.when(g0 + 2 < nch2)
        def _():
            stage_gather(g0 + 2, 0)
        b_cur = process(g0 + 1, 1, b_cur)
        return b_cur

    lax.fori_loop(0, nch2 // 2, pair_body, jnp.int32(0))

    for s in range(2):
        pltpu.make_async_copy(row_bufs[s].at[pl.ds(0, GSUB)],
                              acc_sh.at[seg_bufs[s].at[0]], sems[s]).wait()
        pltpu.make_async_copy(row_bufs[s].at[pl.ds(GSUB, GSUB)],
                              acc_sh.at[seg_bufs[s].at[1]], sems[s]).wait()
    pltpu.sync_copy(acc_sh.at[pl.ds(sbase, BPW)],
                    out_hbm.at[pl.ds(obase, BPW)])


_pooled = functools.partial(
    pl.kernel,
    out_type=jax.ShapeDtypeStruct((BAGS, DIM), jnp.float32),
    mesh=plsc.VectorSubcoreMesh(core_axis_name="c", subcore_axis_name="s"),
    compiler_params=pltpu.CompilerParams(use_tc_tiling_on_sc=False,
                                         needs_layout_passes=False),
    scratch_types=[
        pltpu.VMEM((BPW + 24,), jnp.int32),
        pltpu.VMEM((GSUB,), jnp.int32),
        pltpu.VMEM((GSUB,), jnp.int32),
        pltpu.VMEM((GSUB,), jnp.int32),
        pltpu.VMEM((GSUB,), jnp.int32),
        pltpu.VMEM((C, DIM), jnp.float32),
        pltpu.VMEM((C, DIM), jnp.float32),
        pltpu.VMEM((C,), jnp.int32),
        pltpu.VMEM((2, GSUB), jnp.int32),
        pltpu.VMEM((2, GSUB), jnp.int32),
        pltpu.VMEM_SHARED((NUM_SUBCORES * SLAB, DIM), jnp.float32),
        pltpu.SemaphoreType.DMA,
        pltpu.SemaphoreType.DMA,
        pltpu.SemaphoreType.DMA,
        pltpu.SemaphoreType.DMA,
        pltpu.SemaphoreType.DMA,
        pltpu.SemaphoreType.DMA,
        pltpu.SemaphoreType.DMA,
    ],
)(_pooled_body)


def kernel(indices, offsets, table):
    idx = indices.astype(jnp.int32)
    off = offsets.astype(jnp.int32)
    idx = jnp.concatenate([idx, jnp.zeros((2 * C + 8,), jnp.int32)])
    off = jnp.concatenate([off, jnp.full((24,), N_IDX, jnp.int32)])
    return _pooled(idx, off, table)
